# baseline (speedup 1.0000x reference)
import functools

import jax
import jax.numpy as jnp
from jax.experimental import pallas as pl
from jax.experimental.pallas import tpu as pltpu

TM = 1024
TM0 = 512


def _epilogue(acc, hi, wl, wr, b, act):
    agg = acc.astype(jnp.bfloat16)
    out = (jnp.dot(agg, wl, preferred_element_type=jnp.float32)
           + jnp.dot(hi, wr, preferred_element_type=jnp.float32)
           + b)
    if act == "relu":
        out = jnp.maximum(out, 0.0)
    elif act == "log_softmax":
        m = jnp.max(out, axis=-1, keepdims=True)
        s = out - m
        out = s - jnp.log(jnp.sum(jnp.exp(s), axis=-1, keepdims=True))
    return out


def _layer0_kernel(adj_ref, x_ref, wl_ref, wr_ref, b_ref,
                   o_ref, adjb_ref, *, tm, act):
    i = pl.program_id(0)
    ab = adj_ref[...].astype(jnp.bfloat16)
    adjb_ref[...] = ab
    xb = x_ref[...].astype(jnp.bfloat16)
    acc = jnp.dot(ab, xb, preferred_element_type=jnp.float32)
    hi = x_ref[pl.ds(i * tm, tm), :].astype(jnp.bfloat16)
    out = _epilogue(acc, hi,
                    wl_ref[...].astype(jnp.bfloat16),
                    wr_ref[...].astype(jnp.bfloat16),
                    b_ref[...], act)
    o_ref[...] = out.astype(o_ref.dtype)


def _layer_kernel(adjl_ref, adjr_ref, h_ref, wl_ref, wr_ref, b_ref, o_ref,
                  *, tm, act):
    i = pl.program_id(0)
    nh = adjl_ref.shape[1]
    acc = (jnp.dot(adjl_ref[...], h_ref[pl.ds(0, nh), :],
                   preferred_element_type=jnp.float32)
           + jnp.dot(adjr_ref[...], h_ref[pl.ds(nh, nh), :],
                     preferred_element_type=jnp.float32))
    hi = h_ref[pl.ds(i * tm, tm), :]
    out = _epilogue(acc, hi,
                    wl_ref[...].astype(jnp.bfloat16),
                    wr_ref[...].astype(jnp.bfloat16),
                    b_ref[...], act)
    o_ref[...] = out.astype(o_ref.dtype)


def _sage_layer0(adj_f32, x, wl, wr, b, *, act, out_dtype):
    n = adj_f32.shape[0]
    f_in = x.shape[1]
    f_out = wl.shape[1]
    tm = TM0
    return pl.pallas_call(
        functools.partial(_layer0_kernel, tm=tm, act=act),
        out_shape=(jax.ShapeDtypeStruct((n, f_out), out_dtype),
                   jax.ShapeDtypeStruct((n, n), jnp.bfloat16)),
        grid_spec=pltpu.PrefetchScalarGridSpec(
            num_scalar_prefetch=0,
            grid=(n // tm,),
            in_specs=[
                pl.BlockSpec((tm, n), lambda i: (i, 0)),
                pl.BlockSpec((n, f_in), lambda i: (0, 0)),
                pl.BlockSpec((f_in, f_out), lambda i: (0, 0)),
                pl.BlockSpec((f_in, f_out), lambda i: (0, 0)),
                pl.BlockSpec((1, f_out), lambda i: (0, 0)),
            ],
            out_specs=(pl.BlockSpec((tm, f_out), lambda i: (i, 0)),
                       pl.BlockSpec((tm, n), lambda i: (i, 0))),
        ),
        compiler_params=pltpu.CompilerParams(
            dimension_semantics=("parallel",)),
    )(adj_f32, x, wl, wr, b)


def _sage_layer(adj_b, h, wl, wr, b, *, act, out_dtype):
    n = adj_b.shape[0]
    f_in = h.shape[1]
    f_out = wl.shape[1]
    tm = TM
    return pl.pallas_call(
        functools.partial(_layer_kernel, tm=tm, act=act),
        out_shape=jax.ShapeDtypeStruct((n, f_out), out_dtype),
        grid_spec=pltpu.PrefetchScalarGridSpec(
            num_scalar_prefetch=0,
            grid=(n // tm,),
            in_specs=[
                pl.BlockSpec((tm, n // 2), lambda i: (i, 0)),
                pl.BlockSpec((tm, n // 2), lambda i: (i, 1)),
                pl.BlockSpec((n, f_in), lambda i: (0, 0)),
                pl.BlockSpec((f_in, f_out), lambda i: (0, 0)),
                pl.BlockSpec((f_in, f_out), lambda i: (0, 0)),
                pl.BlockSpec((1, f_out), lambda i: (0, 0)),
            ],
            out_specs=pl.BlockSpec((tm, f_out), lambda i: (i, 0)),
        ),
        compiler_params=pltpu.CompilerParams(
            dimension_semantics=("parallel",)),
    )(adj_b, adj_b, h, wl, wr, b)


def kernel(x, adj, w_l_0, w_r_0, b_0, w_l_1, w_r_1, b_1, w_l_2, w_r_2, b_2):
    h, adj_b = _sage_layer0(adj, x, w_l_0, w_r_0, b_0,
                            act="relu", out_dtype=jnp.bfloat16)
    h = _sage_layer(adj_b, h, w_l_1, w_r_1, b_1,
                    act="relu", out_dtype=jnp.bfloat16)
    out = _sage_layer(adj_b, h, w_l_2, w_r_2, b_2,
                      act="log_softmax", out_dtype=jnp.float32)
    return out


# speedup vs baseline: 1.0059x; 1.0059x over previous
import functools

import jax
import jax.numpy as jnp
from jax.experimental import pallas as pl
from jax.experimental.pallas import tpu as pltpu

TM = 1024
TM0 = 512


def _epilogue(acc, hi, wl, wr, b, act):
    agg = acc.astype(jnp.bfloat16)
    out = (jnp.dot(agg, wl, preferred_element_type=jnp.float32)
           + jnp.dot(hi, wr, preferred_element_type=jnp.float32)
           + b)
    if act == "relu":
        out = jnp.maximum(out, 0.0)
    elif act == "log_softmax":
        m = jnp.max(out, axis=-1, keepdims=True)
        s = out - m
        out = s - jnp.log(jnp.sum(jnp.exp(s), axis=-1, keepdims=True))
    return out


def _layer0_kernel(adj_ref, x_ref, wl_ref, wr_ref, b_ref,
                   o_ref, adjb_ref, *, tm, act):
    i = pl.program_id(0)
    ab = adj_ref[...].astype(jnp.bfloat16)
    adjb_ref[...] = ab
    xb = x_ref[...].astype(jnp.bfloat16)
    acc = jnp.dot(ab, xb, preferred_element_type=jnp.float32)
    hi = x_ref[pl.ds(i * tm, tm), :].astype(jnp.bfloat16)
    out = _epilogue(acc, hi,
                    wl_ref[...].astype(jnp.bfloat16),
                    wr_ref[...].astype(jnp.bfloat16),
                    b_ref[...], act)
    o_ref[...] = out.astype(o_ref.dtype)


def _layer_kernel(adj_ref, h_ref, wl_ref, wr_ref, b_ref, o_ref, *, tm, act):
    i = pl.program_id(0)
    acc = jnp.dot(adj_ref[...], h_ref[...],
                  preferred_element_type=jnp.float32)
    hi = h_ref[pl.ds(i * tm, tm), :]
    out = _epilogue(acc, hi,
                    wl_ref[...].astype(jnp.bfloat16),
                    wr_ref[...].astype(jnp.bfloat16),
                    b_ref[...], act)
    o_ref[...] = out.astype(o_ref.dtype)


def _sage_layer0(adj_f32, x, wl, wr, b, *, act, out_dtype):
    n = adj_f32.shape[0]
    f_in = x.shape[1]
    f_out = wl.shape[1]
    tm = TM0
    return pl.pallas_call(
        functools.partial(_layer0_kernel, tm=tm, act=act),
        out_shape=(jax.ShapeDtypeStruct((n, f_out), out_dtype),
                   jax.ShapeDtypeStruct((n, n), jnp.bfloat16)),
        grid_spec=pltpu.PrefetchScalarGridSpec(
            num_scalar_prefetch=0,
            grid=(n // tm,),
            in_specs=[
                pl.BlockSpec((tm, n), lambda i: (i, 0)),
                pl.BlockSpec((n, f_in), lambda i: (0, 0)),
                pl.BlockSpec((f_in, f_out), lambda i: (0, 0)),
                pl.BlockSpec((f_in, f_out), lambda i: (0, 0)),
                pl.BlockSpec((1, f_out), lambda i: (0, 0)),
            ],
            out_specs=(pl.BlockSpec((tm, f_out), lambda i: (i, 0)),
                       pl.BlockSpec((tm, n), lambda i: (i, 0))),
        ),
        compiler_params=pltpu.CompilerParams(
            dimension_semantics=("parallel",)),
    )(adj_f32, x, wl, wr, b)


def _sage_layer(adj_b, h, wl, wr, b, *, act, out_dtype):
    n = adj_b.shape[0]
    f_in = h.shape[1]
    f_out = wl.shape[1]
    tm = TM
    return pl.pallas_call(
        functools.partial(_layer_kernel, tm=tm, act=act),
        out_shape=jax.ShapeDtypeStruct((n, f_out), out_dtype),
        grid_spec=pltpu.PrefetchScalarGridSpec(
            num_scalar_prefetch=0,
            grid=(n // tm,),
            in_specs=[
                pl.BlockSpec((tm, n), lambda i: (i, 0)),
                pl.BlockSpec((n, f_in), lambda i: (0, 0)),
                pl.BlockSpec((f_in, f_out), lambda i: (0, 0)),
                pl.BlockSpec((f_in, f_out), lambda i: (0, 0)),
                pl.BlockSpec((1, f_out), lambda i: (0, 0)),
            ],
            out_specs=pl.BlockSpec((tm, f_out), lambda i: (i, 0)),
        ),
        compiler_params=pltpu.CompilerParams(
            dimension_semantics=("parallel",)),
    )(adj_b, h, wl, wr, b)


def kernel(x, adj, w_l_0, w_r_0, b_0, w_l_1, w_r_1, b_1, w_l_2, w_r_2, b_2):
    h, adj_b = _sage_layer0(adj, x, w_l_0, w_r_0, b_0,
                            act="relu", out_dtype=jnp.bfloat16)
    h = _sage_layer(adj_b, h, w_l_1, w_r_1, b_1,
                    act="relu", out_dtype=jnp.bfloat16)
    out = _sage_layer(adj_b, h, w_l_2, w_r_2, b_2,
                      act="log_softmax", out_dtype=jnp.float32)
    return out
